# Initial kernel scaffold; baseline (speedup 1.0000x reference)
import jax
import jax.numpy as jnp
from jax.experimental import pallas as pl


def kernel(x, edge_index, W1, b1, W2, b2):
    raise NotImplementedError("write your pallas kernel here")


# speedup vs baseline: 16.2089x; 16.2089x over previous
import functools

import jax
import jax.numpy as jnp
from jax import lax
from jax.experimental import pallas as pl
from jax.experimental.pallas import tpu as pltpu
from jax.experimental.pallas import tpu_sc as plsc

_NC = 2
_NS = 16
_NW = _NC * _NS
_CHUNK = 128
_DEG_W = 16
_RBLK = 80


def _deg_count(dst, n_nodes):
  n_edges = dst.shape[0]
  n_chunks = n_edges // _CHUNK
  iters = (n_chunks + _NW - 1) // _NW
  n_blk = n_nodes // _RBLK
  blk_iters = (n_blk + _NS - 1) // _NS
  mesh = plsc.VectorSubcoreMesh(core_axis_name="c", subcore_axis_name="s")
  ones = jnp.ones((_CHUNK, _DEG_W), jnp.float32)
  zeros = jnp.zeros((_RBLK, _DEG_W), jnp.float32)

  @functools.partial(
      pl.kernel,
      out_type=jax.ShapeDtypeStruct((_NC, n_nodes, _DEG_W), jnp.float32),
      mesh=mesh,
      compiler_params=pltpu.CompilerParams(use_tc_tiling_on_sc=False),
      scratch_types=[
          pltpu.VMEM((_CHUNK,), jnp.int32),
          pltpu.VMEM((_CHUNK, _DEG_W), jnp.float32),
          pltpu.VMEM((_RBLK, _DEG_W), jnp.float32),
          pltpu.VMEM_SHARED((n_nodes, _DEG_W), jnp.float32),
      ],
  )
  def k(dst_h, ones_h, z_h, out_h, dst_v, ones_v, stage_v, acc):
    cid = lax.axis_index("c")
    sid = lax.axis_index("s")
    wid = sid * _NC + cid
    pltpu.sync_copy(ones_h, ones_v)

    def zbody(i, carry):
      b = sid + i * _NS

      @pl.when(b < n_blk)
      def _():
        pltpu.sync_copy(z_h, acc.at[pl.ds(b * _RBLK, _RBLK), :])

      return carry

    lax.fori_loop(0, blk_iters, zbody, 0)
    plsc.subcore_barrier()

    def body(i, carry):
      c = wid + i * _NW

      @pl.when(c < n_chunks)
      def _():
        pltpu.sync_copy(dst_h.at[pl.ds(c * _CHUNK, _CHUNK)], dst_v)
        pltpu.sync_copy(ones_v, acc.at[dst_v], add=True)

      return carry

    lax.fori_loop(0, iters, body, 0)
    plsc.subcore_barrier()

    def obody(i, carry):
      b = sid + i * _NS

      @pl.when(b < n_blk)
      def _():
        pltpu.sync_copy(acc.at[pl.ds(b * _RBLK, _RBLK), :], stage_v)
        pltpu.sync_copy(stage_v, out_h.at[cid, pl.ds(b * _RBLK, _RBLK), :])

      return carry

    lax.fori_loop(0, blk_iters, obody, 0)

  return k(dst, ones, zeros)


def _edge_agg(src, dst, table, n_nodes):
  n_edges = src.shape[0]
  d = table.shape[1]
  n_chunks = n_edges // _CHUNK
  iters = (n_chunks + _NW - 1) // _NW
  n_blk = n_nodes // _RBLK
  blk_iters = (n_blk + _NS - 1) // _NS
  mesh = plsc.VectorSubcoreMesh(core_axis_name="c", subcore_axis_name="s")
  zeros = jnp.zeros((_RBLK, d), jnp.float32)

  @functools.partial(
      pl.kernel,
      out_type=jax.ShapeDtypeStruct((_NC, n_nodes, d), jnp.float32),
      mesh=mesh,
      compiler_params=pltpu.CompilerParams(use_tc_tiling_on_sc=False),
      scratch_types=[
          pltpu.VMEM((_CHUNK,), jnp.int32),
          pltpu.VMEM((_CHUNK,), jnp.int32),
          pltpu.VMEM((_CHUNK, d), jnp.float32),
          pltpu.VMEM((_RBLK, d), jnp.float32),
          pltpu.VMEM_SHARED((n_nodes, d), jnp.float32),
          pltpu.SemaphoreType.DMA,
      ],
  )
  def k(src_h, dst_h, tab_h, z_h, out_h, src_v, dst_v, rows_v, stage_v, acc,
        sem):
    cid = lax.axis_index("c")
    sid = lax.axis_index("s")
    wid = sid * _NC + cid

    def zbody(i, carry):
      b = sid + i * _NS

      @pl.when(b < n_blk)
      def _():
        pltpu.sync_copy(z_h, acc.at[pl.ds(b * _RBLK, _RBLK), :])

      return carry

    lax.fori_loop(0, blk_iters, zbody, 0)
    plsc.subcore_barrier()

    def body(i, carry):
      c = wid + i * _NW

      @pl.when(c < n_chunks)
      def _():
        off = c * _CHUNK
        pltpu.sync_copy(src_h.at[pl.ds(off, _CHUNK)], src_v)
        pltpu.sync_copy(dst_h.at[pl.ds(off, _CHUNK)], dst_v)
        pltpu.async_copy(tab_h.at[src_v], rows_v, sem).wait()
        pltpu.sync_copy(rows_v, acc.at[dst_v], add=True)

      return carry

    lax.fori_loop(0, iters, body, 0)
    plsc.subcore_barrier()

    def obody(i, carry):
      b = sid + i * _NS

      @pl.when(b < n_blk)
      def _():
        pltpu.sync_copy(acc.at[pl.ds(b * _RBLK, _RBLK), :], stage_v)
        pltpu.sync_copy(stage_v, out_h.at[cid, pl.ds(b * _RBLK, _RBLK), :])

      return carry

    lax.fori_loop(0, blk_iters, obody, 0)

  return k(src, dst, table, zeros)


def _tc_matmul(x, w):
  def body(x_ref, w_ref, o_ref):
    o_ref[...] = jnp.dot(x_ref[...], w_ref[...],
                         preferred_element_type=jnp.float32)

  return pl.pallas_call(
      body,
      out_shape=jax.ShapeDtypeStruct((x.shape[0], w.shape[1]), jnp.float32),
  )(x, w)


def _tc_dinv_scale(degp, xw):
  n = xw.shape[0]

  def body(d_ref, xw_ref, dinv_ref, y_ref):
    dsum = d_ref[0] + d_ref[1]
    deg = dsum[:, 0:1] + 1.0
    dinv = lax.rsqrt(deg)
    dinv_ref[...] = dinv
    y_ref[...] = xw_ref[...] * dinv

  return pl.pallas_call(
      body,
      out_shape=(
          jax.ShapeDtypeStruct((n, 1), jnp.float32),
          jax.ShapeDtypeStruct(xw.shape, jnp.float32),
      ),
  )(degp, xw)


def _tc_mid(accp, y1, dinv, b1, w2):
  n = y1.shape[0]

  def body(a_ref, y_ref, di_ref, b_ref, w_ref, o_ref):
    di = di_ref[...]
    s = a_ref[0] + a_ref[1] + y_ref[...]
    h = jnp.maximum(di * s + b_ref[...], 0.0)
    o_ref[...] = di * jnp.dot(h, w_ref[...],
                              preferred_element_type=jnp.float32)

  return pl.pallas_call(
      body,
      out_shape=jax.ShapeDtypeStruct((n, w2.shape[1]), jnp.float32),
  )(accp, y1, dinv, b1, w2)


def _tc_final(accp, y2, dinv, b2):
  def body(a_ref, y_ref, di_ref, b_ref, o_ref):
    s = a_ref[0] + a_ref[1] + y_ref[...]
    o_ref[...] = di_ref[...] * s + b_ref[...]

  return pl.pallas_call(
      body,
      out_shape=jax.ShapeDtypeStruct(y2.shape, jnp.float32),
  )(accp, y2, dinv, b2)


def kernel(x, edge_index, W1, b1, W2, b2):
  n = x.shape[0]
  src = edge_index[0].astype(jnp.int32)
  dst = edge_index[1].astype(jnp.int32)

  degp = _deg_count(dst, n)
  xw1 = _tc_matmul(x, W1)
  dinv, y1 = _tc_dinv_scale(degp, xw1)
  acc1 = _edge_agg(src, dst, y1, n)
  y2 = _tc_mid(acc1, y1, dinv, b1.reshape(1, -1), W2)
  acc2 = _edge_agg(src, dst, y2, n)
  out = _tc_final(acc2, y2, dinv, b2.reshape(1, -1))
  return out


# speedup vs baseline: 25.9908x; 1.6035x over previous
import functools

import jax
import jax.numpy as jnp
from jax import lax
from jax.experimental import pallas as pl
from jax.experimental.pallas import tpu as pltpu
from jax.experimental.pallas import tpu_sc as plsc

_NC = 2
_NS = 16
_NW = _NC * _NS
_CHUNK = 128
_GRP = 8
_DEG_W = 16
_RBLK = 80
_PAD = 8


def _deg_count(dst2d, n_nodes):
  n_groups = dst2d.shape[0] // _GRP
  iters = (n_groups + _NW - 1) // _NW
  n_blk = n_nodes // _RBLK
  blk_iters = (n_blk + _NS - 1) // _NS
  mesh = plsc.VectorSubcoreMesh(core_axis_name="c", subcore_axis_name="s")
  ones = jnp.ones((_CHUNK, _DEG_W), jnp.float32)
  zeros = jnp.zeros((_RBLK, _DEG_W), jnp.float32)

  @functools.partial(
      pl.kernel,
      out_type=jax.ShapeDtypeStruct((_NC, n_nodes, _DEG_W), jnp.float32),
      mesh=mesh,
      compiler_params=pltpu.CompilerParams(use_tc_tiling_on_sc=False),
      scratch_types=[
          pltpu.VMEM((_GRP, _CHUNK), jnp.int32),
          pltpu.VMEM((_CHUNK, _DEG_W), jnp.float32),
          pltpu.VMEM((_RBLK, _DEG_W), jnp.float32),
          pltpu.VMEM_SHARED((n_nodes + _PAD, _DEG_W), jnp.float32),
          pltpu.SemaphoreType.DMA,
      ],
  )
  def k(dst_h, ones_h, z_h, out_h, dst_v, ones_v, stage_v, acc, sem):
    cid = lax.axis_index("c")
    sid = lax.axis_index("s")
    wid = sid * _NC + cid
    pltpu.sync_copy(ones_h, ones_v)

    def zbody(i, carry):
      b = sid + i * _NS

      @pl.when(b < n_blk)
      def _():
        pltpu.sync_copy(z_h, acc.at[pl.ds(b * _RBLK, _RBLK), :])

      return carry

    lax.fori_loop(0, blk_iters, zbody, 0)
    plsc.subcore_barrier()

    def body(i, carry):
      g = wid + i * _NW

      @pl.when(g < n_groups)
      def _():
        pltpu.sync_copy(dst_h.at[pl.ds(g * _GRP, _GRP), :], dst_v)
        descs = [
            pltpu.async_copy(ones_v, acc.at[dst_v.at[j]], sem, add=True)
            for j in range(_GRP)
        ]
        for d_ in descs:
          d_.wait()

      return carry

    lax.fori_loop(0, iters, body, 0)
    plsc.subcore_barrier()

    def obody(i, carry):
      b = sid + i * _NS

      @pl.when(b < n_blk)
      def _():
        pltpu.sync_copy(acc.at[pl.ds(b * _RBLK, _RBLK), :], stage_v)
        pltpu.sync_copy(stage_v, out_h.at[cid, pl.ds(b * _RBLK, _RBLK), :])

      return carry

    lax.fori_loop(0, blk_iters, obody, 0)

  return k(dst2d, ones, zeros)


def _edge_agg(src2d, dst2d, table, n_nodes):
  d = table.shape[1]
  n_groups = src2d.shape[0] // _GRP
  iters = (n_groups + _NW - 1) // _NW
  n_blk = n_nodes // _RBLK
  blk_iters = (n_blk + _NS - 1) // _NS
  mesh = plsc.VectorSubcoreMesh(core_axis_name="c", subcore_axis_name="s")
  zeros = jnp.zeros((_RBLK, d), jnp.float32)

  @functools.partial(
      pl.kernel,
      out_type=jax.ShapeDtypeStruct((_NC, n_nodes, d), jnp.float32),
      mesh=mesh,
      compiler_params=pltpu.CompilerParams(use_tc_tiling_on_sc=False),
      scratch_types=[
          pltpu.VMEM((_GRP, _CHUNK), jnp.int32),
          pltpu.VMEM((_GRP, _CHUNK), jnp.int32),
          pltpu.VMEM((_CHUNK, d), jnp.float32),
          pltpu.VMEM((_CHUNK, d), jnp.float32),
          pltpu.VMEM((_RBLK, d), jnp.float32),
          pltpu.VMEM_SHARED((n_nodes + _PAD, d), jnp.float32),
          pltpu.SemaphoreType.DMA,
          pltpu.SemaphoreType.DMA,
      ],
  )
  def k(src_h, dst_h, tab_h, z_h, out_h, src_v, dst_v, rows_a, rows_b,
        stage_v, acc, sem_a, sem_b):
    cid = lax.axis_index("c")
    sid = lax.axis_index("s")
    wid = sid * _NC + cid
    rows = (rows_a, rows_b)
    sems = (sem_a, sem_b)

    def zbody(i, carry):
      b = sid + i * _NS

      @pl.when(b < n_blk)
      def _():
        pltpu.sync_copy(z_h, acc.at[pl.ds(b * _RBLK, _RBLK), :])

      return carry

    lax.fori_loop(0, blk_iters, zbody, 0)
    plsc.subcore_barrier()

    def body(i, carry):
      g = wid + i * _NW

      @pl.when(g < n_groups)
      def _():
        pltpu.sync_copy(src_h.at[pl.ds(g * _GRP, _GRP), :], src_v)
        pltpu.sync_copy(dst_h.at[pl.ds(g * _GRP, _GRP), :], dst_v)
        descs = [None] * _GRP
        descs[0] = pltpu.async_copy(tab_h.at[src_v.at[0]], rows[0], sems[0])
        for j in range(1, _GRP + 1):
          if j < _GRP:
            descs[j] = pltpu.async_copy(
                tab_h.at[src_v.at[j]], rows[j % 2], sems[j % 2])
          descs[j - 1].wait()
          pltpu.sync_copy(rows[(j - 1) % 2], acc.at[dst_v.at[j - 1]],
                          add=True)

      return carry

    lax.fori_loop(0, iters, body, 0)
    plsc.subcore_barrier()

    def obody(i, carry):
      b = sid + i * _NS

      @pl.when(b < n_blk)
      def _():
        pltpu.sync_copy(acc.at[pl.ds(b * _RBLK, _RBLK), :], stage_v)
        pltpu.sync_copy(stage_v, out_h.at[cid, pl.ds(b * _RBLK, _RBLK), :])

      return carry

    lax.fori_loop(0, blk_iters, obody, 0)

  return k(src2d, dst2d, table, zeros)


def _tc_matmul(x, w):
  def body(x_ref, w_ref, o_ref):
    o_ref[...] = jnp.dot(x_ref[...], w_ref[...],
                         preferred_element_type=jnp.float32)

  return pl.pallas_call(
      body,
      out_shape=jax.ShapeDtypeStruct((x.shape[0], w.shape[1]), jnp.float32),
  )(x, w)


def _tc_dinv_scale(degp, xw):
  n = xw.shape[0]

  def body(d_ref, xw_ref, dinv_ref, y_ref):
    dsum = d_ref[0] + d_ref[1]
    deg = dsum[:, 0:1] + 1.0
    dinv = lax.rsqrt(deg)
    dinv_ref[...] = dinv
    y_ref[...] = xw_ref[...] * dinv

  return pl.pallas_call(
      body,
      out_shape=(
          jax.ShapeDtypeStruct((n, 1), jnp.float32),
          jax.ShapeDtypeStruct(xw.shape, jnp.float32),
      ),
  )(degp, xw)


def _tc_mid(accp, y1, dinv, b1, w2):
  n = y1.shape[0]

  def body(a_ref, y_ref, di_ref, b_ref, w_ref, o_ref):
    di = di_ref[...]
    s = a_ref[0] + a_ref[1] + y_ref[...]
    h = jnp.maximum(di * s + b_ref[...], 0.0)
    o_ref[...] = di * jnp.dot(h, w_ref[...],
                              preferred_element_type=jnp.float32)

  return pl.pallas_call(
      body,
      out_shape=jax.ShapeDtypeStruct((n, w2.shape[1]), jnp.float32),
  )(accp, y1, dinv, b1, w2)


def _tc_final(accp, y2, dinv, b2):
  def body(a_ref, y_ref, di_ref, b_ref, o_ref):
    s = a_ref[0] + a_ref[1] + y_ref[...]
    o_ref[...] = di_ref[...] * s + b_ref[...]

  return pl.pallas_call(
      body,
      out_shape=jax.ShapeDtypeStruct(y2.shape, jnp.float32),
  )(accp, y2, dinv, b2)


def kernel(x, edge_index, W1, b1, W2, b2):
  n = x.shape[0]
  n_edges = edge_index.shape[1]
  grp_edges = _GRP * _CHUNK
  n_pad = (-n_edges) % grp_edges
  src = edge_index[0].astype(jnp.int32)
  dst = edge_index[1].astype(jnp.int32)
  src = jnp.concatenate([src, jnp.zeros((n_pad,), jnp.int32)])
  dst = jnp.concatenate([dst, jnp.full((n_pad,), n, jnp.int32)])
  src = src.reshape(-1, _CHUNK)
  dst = dst.reshape(-1, _CHUNK)

  degp = _deg_count(dst, n)
  xw1 = _tc_matmul(x, W1)
  dinv, y1 = _tc_dinv_scale(degp, xw1)
  acc1 = _edge_agg(src, dst, y1, n)
  y2 = _tc_mid(acc1, y1, dinv, b1.reshape(1, -1), W2)
  acc2 = _edge_agg(src, dst, y2, n)
  out = _tc_final(acc2, y2, dinv, b2.reshape(1, -1))
  return out
